# Initial kernel scaffold; baseline (speedup 1.0000x reference)
import jax
import jax.numpy as jnp
from jax.experimental import pallas as pl


def kernel(x, adj_norm, W0, W1, W2):
    raise NotImplementedError("write your pallas kernel here")


# speedup vs baseline: 1.0712x; 1.0712x over previous
import jax
import jax.numpy as jnp
from jax.experimental import pallas as pl

N = 8192
D = 128
BM = 256
BY = 1024


def _xw_kernel(h_ref, w_ref, y_ref):
    y = jnp.dot(h_ref[...], w_ref[...], preferred_element_type=jnp.float32)
    y_ref[...] = y.astype(jnp.bfloat16)


def _xw(h, w):
    return pl.pallas_call(
        _xw_kernel,
        grid=(N // BY,),
        in_specs=[
            pl.BlockSpec((BY, D), lambda i: (i, 0)),
            pl.BlockSpec((D, D), lambda i: (0, 0)),
        ],
        out_specs=pl.BlockSpec((BY, D), lambda i: (i, 0)),
        out_shape=jax.ShapeDtypeStruct((N, D), jnp.bfloat16),
    )(h, w)


def _layer0_kernel(a_ref, y_ref, h_ref, a16_ref):
    a16 = a_ref[...].astype(jnp.bfloat16)
    a16_ref[...] = a16
    acc = jnp.dot(a16, y_ref[...], preferred_element_type=jnp.float32)
    h_ref[...] = jnp.maximum(acc, 0.0)


def _layer0(a, y):
    return pl.pallas_call(
        _layer0_kernel,
        grid=(N // BM,),
        in_specs=[
            pl.BlockSpec((BM, N), lambda i: (i, 0)),
            pl.BlockSpec((N, D), lambda i: (0, 0)),
        ],
        out_specs=[
            pl.BlockSpec((BM, D), lambda i: (i, 0)),
            pl.BlockSpec((BM, N), lambda i: (i, 0)),
        ],
        out_shape=[
            jax.ShapeDtypeStruct((N, D), jnp.float32),
            jax.ShapeDtypeStruct((N, N), jnp.bfloat16),
        ],
    )(a, y)


def _layer_kernel(a16_ref, y_ref, h_ref):
    acc = jnp.dot(a16_ref[...], y_ref[...], preferred_element_type=jnp.float32)
    h_ref[...] = jnp.maximum(acc, 0.0)


def _layer(a16, y):
    return pl.pallas_call(
        _layer_kernel,
        grid=(N // BM,),
        in_specs=[
            pl.BlockSpec((BM, N), lambda i: (i, 0)),
            pl.BlockSpec((N, D), lambda i: (0, 0)),
        ],
        out_specs=pl.BlockSpec((BM, D), lambda i: (i, 0)),
        out_shape=jax.ShapeDtypeStruct((N, D), jnp.float32),
    )(a16, y)


def kernel(x, adj_norm, W0, W1, W2):
    y0 = _xw(x, W0)
    h1, a16 = _layer0(adj_norm, y0)
    y1 = _xw(h1, W1)
    h2 = _layer(a16, y1)
    y2 = _xw(h2, W2)
    h3 = _layer(a16, y2)
    return jnp.concatenate([h1, h2, h3], axis=-1)


# speedup vs baseline: 1.2264x; 1.1449x over previous
import jax
import jax.numpy as jnp
from jax.experimental import pallas as pl
from jax.experimental.pallas import tpu as pltpu

N = 8192
D = 128
BM0 = 256
BM = 512


def _layer0_kernel(a_ref, x_ref, w0_ref, w1_ref, h_ref, a16_ref, y1_ref, y0_scr):
    @pl.when(pl.program_id(0) == 0)
    def _prologue():
        y0 = jnp.dot(x_ref[...], w0_ref[...], preferred_element_type=jnp.float32)
        y0_scr[...] = y0.astype(jnp.bfloat16)

    a16 = a_ref[...].astype(jnp.bfloat16)
    a16_ref[...] = a16
    h = jnp.maximum(
        jnp.dot(a16, y0_scr[...], preferred_element_type=jnp.float32), 0.0)
    h_ref[...] = h
    y1 = jnp.dot(h, w1_ref[...], preferred_element_type=jnp.float32)
    y1_ref[...] = y1.astype(jnp.bfloat16)


def _layer0(a, x, w0, w1):
    return pl.pallas_call(
        _layer0_kernel,
        grid=(N // BM0,),
        in_specs=[
            pl.BlockSpec((BM0, N), lambda i: (i, 0)),
            pl.BlockSpec((N, D), lambda i: (0, 0)),
            pl.BlockSpec((D, D), lambda i: (0, 0)),
            pl.BlockSpec((D, D), lambda i: (0, 0)),
        ],
        out_specs=[
            pl.BlockSpec((BM0, D), lambda i: (i, 0)),
            pl.BlockSpec((BM0, N), lambda i: (i, 0)),
            pl.BlockSpec((BM0, D), lambda i: (i, 0)),
        ],
        out_shape=[
            jax.ShapeDtypeStruct((N, D), jnp.float32),
            jax.ShapeDtypeStruct((N, N), jnp.bfloat16),
            jax.ShapeDtypeStruct((N, D), jnp.bfloat16),
        ],
        scratch_shapes=[pltpu.VMEM((N, D), jnp.bfloat16)],
    )(a, x, w0, w1)


def _layer1_kernel(a16_ref, y_ref, w_ref, h_ref, ynext_ref):
    h = jnp.maximum(
        jnp.dot(a16_ref[...], y_ref[...], preferred_element_type=jnp.float32),
        0.0)
    h_ref[...] = h
    ynext = jnp.dot(h, w_ref[...], preferred_element_type=jnp.float32)
    ynext_ref[...] = ynext.astype(jnp.bfloat16)


def _layer1(a16, y, w):
    return pl.pallas_call(
        _layer1_kernel,
        grid=(N // BM,),
        in_specs=[
            pl.BlockSpec((BM, N), lambda i: (i, 0)),
            pl.BlockSpec((N, D), lambda i: (0, 0)),
            pl.BlockSpec((D, D), lambda i: (0, 0)),
        ],
        out_specs=[
            pl.BlockSpec((BM, D), lambda i: (i, 0)),
            pl.BlockSpec((BM, D), lambda i: (i, 0)),
        ],
        out_shape=[
            jax.ShapeDtypeStruct((N, D), jnp.float32),
            jax.ShapeDtypeStruct((N, D), jnp.bfloat16),
        ],
    )(a16, y, w)


def _layer2_kernel(a16_ref, y_ref, h_ref):
    h_ref[...] = jnp.maximum(
        jnp.dot(a16_ref[...], y_ref[...], preferred_element_type=jnp.float32),
        0.0)


def _layer2(a16, y):
    return pl.pallas_call(
        _layer2_kernel,
        grid=(N // BM,),
        in_specs=[
            pl.BlockSpec((BM, N), lambda i: (i, 0)),
            pl.BlockSpec((N, D), lambda i: (0, 0)),
        ],
        out_specs=pl.BlockSpec((BM, D), lambda i: (i, 0)),
        out_shape=jax.ShapeDtypeStruct((N, D), jnp.float32),
    )(a16, y)


def kernel(x, adj_norm, W0, W1, W2):
    h1, a16, y1 = _layer0(adj_norm, x, W0, W1)
    h2, y2 = _layer1(a16, y1, W2)
    h3 = _layer2(a16, y2)
    return jnp.concatenate([h1, h2, h3], axis=-1)


# speedup vs baseline: 1.2677x; 1.0337x over previous
import jax
import jax.numpy as jnp
from jax.experimental import pallas as pl
from jax.experimental.pallas import tpu as pltpu

N = 8192
D = 128
BM0 = 256
BM = 512


def _layer0_kernel(a_ref, x_ref, w0_ref, w1_ref, o_ref, a16_ref, y1_ref, y0_scr):
    @pl.when(pl.program_id(0) == 0)
    def _prologue():
        y0 = jnp.dot(x_ref[...], w0_ref[...], preferred_element_type=jnp.float32)
        y0_scr[...] = y0.astype(jnp.bfloat16)

    a16 = a_ref[...].astype(jnp.bfloat16)
    a16_ref[...] = a16
    h = jnp.maximum(
        jnp.dot(a16, y0_scr[...], preferred_element_type=jnp.float32), 0.0)
    o_ref[...] = h
    y1 = jnp.dot(h, w1_ref[...], preferred_element_type=jnp.float32)
    y1_ref[...] = y1.astype(jnp.bfloat16)


def _layer0(a, x, w0, w1):
    return pl.pallas_call(
        _layer0_kernel,
        grid=(N // BM0,),
        in_specs=[
            pl.BlockSpec((BM0, N), lambda i: (i, 0)),
            pl.BlockSpec((N, D), lambda i: (0, 0)),
            pl.BlockSpec((D, D), lambda i: (0, 0)),
            pl.BlockSpec((D, D), lambda i: (0, 0)),
        ],
        out_specs=[
            pl.BlockSpec((BM0, D), lambda i: (i, 0)),
            pl.BlockSpec((BM0, N), lambda i: (i, 0)),
            pl.BlockSpec((BM0, D), lambda i: (i, 0)),
        ],
        out_shape=[
            jax.ShapeDtypeStruct((N, 3 * D), jnp.float32),
            jax.ShapeDtypeStruct((N, N), jnp.bfloat16),
            jax.ShapeDtypeStruct((N, D), jnp.bfloat16),
        ],
        scratch_shapes=[pltpu.VMEM((N, D), jnp.bfloat16)],
    )(a, x, w0, w1)


def _layer1_kernel(a16_ref, y_ref, w_ref, o_in_ref, o_ref, ynext_ref):
    del o_in_ref
    h = jnp.maximum(
        jnp.dot(a16_ref[...], y_ref[...], preferred_element_type=jnp.float32),
        0.0)
    o_ref[...] = h
    ynext = jnp.dot(h, w_ref[...], preferred_element_type=jnp.float32)
    ynext_ref[...] = ynext.astype(jnp.bfloat16)


def _layer1(a16, y, w, o):
    return pl.pallas_call(
        _layer1_kernel,
        grid=(N // BM,),
        in_specs=[
            pl.BlockSpec((BM, N), lambda i: (i, 0)),
            pl.BlockSpec((N, D), lambda i: (0, 0)),
            pl.BlockSpec((D, D), lambda i: (0, 0)),
            pl.BlockSpec(memory_space=pl.ANY),
        ],
        out_specs=[
            pl.BlockSpec((BM, D), lambda i: (i, 1)),
            pl.BlockSpec((BM, D), lambda i: (i, 0)),
        ],
        out_shape=[
            jax.ShapeDtypeStruct((N, 3 * D), jnp.float32),
            jax.ShapeDtypeStruct((N, D), jnp.bfloat16),
        ],
        input_output_aliases={3: 0},
    )(a16, y, w, o)


def _layer2_kernel(a16_ref, y_ref, o_in_ref, o_ref):
    del o_in_ref
    o_ref[...] = jnp.maximum(
        jnp.dot(a16_ref[...], y_ref[...], preferred_element_type=jnp.float32),
        0.0)


def _layer2(a16, y, o):
    return pl.pallas_call(
        _layer2_kernel,
        grid=(N // BM,),
        in_specs=[
            pl.BlockSpec((BM, N), lambda i: (i, 0)),
            pl.BlockSpec((N, D), lambda i: (0, 0)),
            pl.BlockSpec(memory_space=pl.ANY),
        ],
        out_specs=pl.BlockSpec((BM, D), lambda i: (i, 2)),
        out_shape=jax.ShapeDtypeStruct((N, 3 * D), jnp.float32),
        input_output_aliases={2: 0},
    )(a16, y, o)


def kernel(x, adj_norm, W0, W1, W2):
    o1, a16, y1 = _layer0(adj_norm, x, W0, W1)
    o2, y2 = _layer1(a16, y1, W2, o1)
    return _layer2(a16, y2, o2)


# speedup vs baseline: 1.3038x; 1.0284x over previous
import jax
import jax.numpy as jnp
from jax.experimental import pallas as pl
from jax.experimental.pallas import tpu as pltpu

N = 8192
D = 128
BM0 = 256
BM = 1024


def _layer0_kernel(a_ref, x_ref, w0_ref, w1_ref, o_ref, a16_ref, y1_ref, y0_scr):
    @pl.when(pl.program_id(0) == 0)
    def _prologue():
        y0 = jnp.dot(x_ref[...], w0_ref[...], preferred_element_type=jnp.float32)
        y0_scr[...] = y0.astype(jnp.bfloat16)

    a16 = a_ref[...].astype(jnp.bfloat16)
    a16_ref[...] = a16
    h = jnp.maximum(
        jnp.dot(a16, y0_scr[...], preferred_element_type=jnp.float32), 0.0)
    o_ref[...] = h
    y1 = jnp.dot(h, w1_ref[...], preferred_element_type=jnp.float32)
    y1_ref[...] = y1.astype(jnp.bfloat16)


def _layer0(a, x, w0, w1):
    return pl.pallas_call(
        _layer0_kernel,
        grid=(N // BM0,),
        in_specs=[
            pl.BlockSpec((BM0, N), lambda i: (i, 0)),
            pl.BlockSpec((N, D), lambda i: (0, 0)),
            pl.BlockSpec((D, D), lambda i: (0, 0)),
            pl.BlockSpec((D, D), lambda i: (0, 0)),
        ],
        out_specs=[
            pl.BlockSpec((BM0, D), lambda i: (i, 0)),
            pl.BlockSpec((BM0, N), lambda i: (i, 0)),
            pl.BlockSpec((BM0, D), lambda i: (i, 0)),
        ],
        out_shape=[
            jax.ShapeDtypeStruct((N, 3 * D), jnp.float32),
            jax.ShapeDtypeStruct((N, N), jnp.bfloat16),
            jax.ShapeDtypeStruct((N, D), jnp.bfloat16),
        ],
        scratch_shapes=[pltpu.VMEM((N, D), jnp.bfloat16)],
    )(a, x, w0, w1)


def _layer1_kernel(a16_ref, y_ref, w_ref, o_in_ref, o_ref, ynext_ref):
    del o_in_ref
    h = jnp.maximum(
        jnp.dot(a16_ref[...], y_ref[...], preferred_element_type=jnp.float32),
        0.0)
    o_ref[...] = h
    ynext = jnp.dot(h, w_ref[...], preferred_element_type=jnp.float32)
    ynext_ref[...] = ynext.astype(jnp.bfloat16)


def _layer1(a16, y, w, o):
    return pl.pallas_call(
        _layer1_kernel,
        grid=(N // BM,),
        in_specs=[
            pl.BlockSpec((BM, N), lambda i: (i, 0)),
            pl.BlockSpec((N, D), lambda i: (0, 0)),
            pl.BlockSpec((D, D), lambda i: (0, 0)),
            pl.BlockSpec(memory_space=pl.ANY),
        ],
        out_specs=[
            pl.BlockSpec((BM, D), lambda i: (i, 1)),
            pl.BlockSpec((BM, D), lambda i: (i, 0)),
        ],
        out_shape=[
            jax.ShapeDtypeStruct((N, 3 * D), jnp.float32),
            jax.ShapeDtypeStruct((N, D), jnp.bfloat16),
        ],
        input_output_aliases={3: 0},
    )(a16, y, w, o)


def _layer2_kernel(a16_ref, y_ref, o_in_ref, o_ref):
    del o_in_ref
    o_ref[...] = jnp.maximum(
        jnp.dot(a16_ref[...], y_ref[...], preferred_element_type=jnp.float32),
        0.0)


def _layer2(a16, y, o):
    return pl.pallas_call(
        _layer2_kernel,
        grid=(N // BM,),
        in_specs=[
            pl.BlockSpec((BM, N), lambda i: (i, 0)),
            pl.BlockSpec((N, D), lambda i: (0, 0)),
            pl.BlockSpec(memory_space=pl.ANY),
        ],
        out_specs=pl.BlockSpec((BM, D), lambda i: (i, 2)),
        out_shape=jax.ShapeDtypeStruct((N, 3 * D), jnp.float32),
        input_output_aliases={2: 0},
    )(a16, y, o)


def kernel(x, adj_norm, W0, W1, W2):
    o1, a16, y1 = _layer0(adj_norm, x, W0, W1)
    o2, y2 = _layer1(a16, y1, W2, o1)
    return _layer2(a16, y2, o2)


# speedup vs baseline: 1.3090x; 1.0040x over previous
import jax
import jax.numpy as jnp
from jax.experimental import pallas as pl
from jax.experimental.pallas import tpu as pltpu

N = 8192
D = 128
BM0 = 512
BM = 1024


def _layer0_kernel(a_ref, x_ref, w0_ref, w1_ref, o_ref, a16_ref, y1_ref, y0_scr):
    @pl.when(pl.program_id(0) == 0)
    def _prologue():
        y0 = jnp.dot(x_ref[...], w0_ref[...], preferred_element_type=jnp.float32)
        y0_scr[...] = y0.astype(jnp.bfloat16)

    a16 = a_ref[...].astype(jnp.bfloat16)
    a16_ref[...] = a16
    h = jnp.maximum(
        jnp.dot(a16, y0_scr[...], preferred_element_type=jnp.float32), 0.0)
    o_ref[...] = h
    y1 = jnp.dot(h, w1_ref[...], preferred_element_type=jnp.float32)
    y1_ref[...] = y1.astype(jnp.bfloat16)


def _layer0(a, x, w0, w1):
    return pl.pallas_call(
        _layer0_kernel,
        grid=(N // BM0,),
        in_specs=[
            pl.BlockSpec((BM0, N), lambda i: (i, 0)),
            pl.BlockSpec((N, D), lambda i: (0, 0)),
            pl.BlockSpec((D, D), lambda i: (0, 0)),
            pl.BlockSpec((D, D), lambda i: (0, 0)),
        ],
        out_specs=[
            pl.BlockSpec((BM0, D), lambda i: (i, 0)),
            pl.BlockSpec((BM0, N), lambda i: (i, 0)),
            pl.BlockSpec((BM0, D), lambda i: (i, 0)),
        ],
        out_shape=[
            jax.ShapeDtypeStruct((N, 3 * D), jnp.float32),
            jax.ShapeDtypeStruct((N, N), jnp.bfloat16),
            jax.ShapeDtypeStruct((N, D), jnp.bfloat16),
        ],
        scratch_shapes=[pltpu.VMEM((N, D), jnp.bfloat16)],
    )(a, x, w0, w1)


def _layer1_kernel(a16_ref, y_ref, w_ref, o_in_ref, o_ref, ynext_ref):
    del o_in_ref
    h = jnp.maximum(
        jnp.dot(a16_ref[...], y_ref[...], preferred_element_type=jnp.float32),
        0.0)
    o_ref[...] = h
    ynext = jnp.dot(h, w_ref[...], preferred_element_type=jnp.float32)
    ynext_ref[...] = ynext.astype(jnp.bfloat16)


def _layer1(a16, y, w, o):
    return pl.pallas_call(
        _layer1_kernel,
        grid=(N // BM,),
        in_specs=[
            pl.BlockSpec((BM, N), lambda i: (i, 0)),
            pl.BlockSpec((N, D), lambda i: (0, 0)),
            pl.BlockSpec((D, D), lambda i: (0, 0)),
            pl.BlockSpec(memory_space=pl.ANY),
        ],
        out_specs=[
            pl.BlockSpec((BM, D), lambda i: (i, 1)),
            pl.BlockSpec((BM, D), lambda i: (i, 0)),
        ],
        out_shape=[
            jax.ShapeDtypeStruct((N, 3 * D), jnp.float32),
            jax.ShapeDtypeStruct((N, D), jnp.bfloat16),
        ],
        input_output_aliases={3: 0},
    )(a16, y, w, o)


def _layer2_kernel(a16_ref, y_ref, o_in_ref, o_ref):
    del o_in_ref
    o_ref[...] = jnp.maximum(
        jnp.dot(a16_ref[...], y_ref[...], preferred_element_type=jnp.float32),
        0.0)


def _layer2(a16, y, o):
    return pl.pallas_call(
        _layer2_kernel,
        grid=(N // BM,),
        in_specs=[
            pl.BlockSpec((BM, N), lambda i: (i, 0)),
            pl.BlockSpec((N, D), lambda i: (0, 0)),
            pl.BlockSpec(memory_space=pl.ANY),
        ],
        out_specs=pl.BlockSpec((BM, D), lambda i: (i, 2)),
        out_shape=jax.ShapeDtypeStruct((N, 3 * D), jnp.float32),
        input_output_aliases={2: 0},
    )(a16, y, o)


def kernel(x, adj_norm, W0, W1, W2):
    o1, a16, y1 = _layer0(adj_norm, x, W0, W1)
    o2, y2 = _layer1(a16, y1, W2, o1)
    return _layer2(a16, y2, o2)
